# baseline (speedup 1.0000x reference)
import jax
import jax.numpy as jnp
from jax.experimental import pallas as pl


def _key_kernel(deg_ref, noise_ref, out_ref):
    out_ref[...] = deg_ref[...] + noise_ref[...]


def kernel(x, edge_index):
    if x.ndim == 2:
        x = x[None, :, :]
    batch, num_nodes, _ = x.shape
    src = edge_index[0]
    deg = jnp.zeros((batch, num_nodes), dtype=x.dtype).at[:, src].add(1.0)
    noise = jax.random.uniform(jax.random.key(1), deg.shape, dtype=x.dtype) * 0.1
    keys = pl.pallas_call(
        _key_kernel,
        out_shape=jax.ShapeDtypeStruct(deg.shape, deg.dtype),
    )(deg, noise)
    sorted_idx = jnp.argsort(keys, axis=-1)
    x_sorted = jnp.take_along_axis(x, sorted_idx[:, :, None], axis=1)
    return (x_sorted, sorted_idx)


# speedup vs baseline: 1.9286x; 1.9286x over previous
import functools

import jax
import jax.numpy as jnp
from jax import lax
from jax.experimental import pallas as pl
from jax.experimental.pallas import tpu as pltpu
from jax.experimental.pallas import tpu_sc as plsc

_N = 10000
_E = 320000
_D = 128
_NP = 10240
_EP = 327680
_PAD_BIN = 10016
_IB = 256
_JC = 1024

def _hist_sc_body(src_hbm, hist_hbm, idx_v, ones_v, zeros_v, hist_sh):
    c = lax.axis_index("c")
    s = lax.axis_index("s")
    w = c * 16 + s
    for k in range(8):
        ones_v[pl.ds(k * 16, 16)] = jnp.full((16,), 1.0, jnp.float32)
    for k in range(40):
        zeros_v[pl.ds(k * 16, 16)] = jnp.zeros((16,), jnp.float32)
    pltpu.sync_copy(zeros_v, hist_sh.at[pl.ds(s * 640, 640)])
    pltpu.sync_copy(src_hbm.at[w], idx_v)
    plsc.subcore_barrier()

    def chunk(j, carry):
        pltpu.sync_copy(ones_v, hist_sh.at[idx_v.at[j]], add=True)
        return carry

    lax.fori_loop(0, 80, chunk, 0)
    plsc.subcore_barrier()
    pltpu.sync_copy(hist_sh.at[pl.ds(s * 640, 640)],
                    hist_hbm.at[c, pl.ds(s * 640, 640)])


def _rank_body(hist_a, hist_b, noise_r, histT, noise_c, out_ref):
    g = pl.program_id(0)
    ki = histT[:, 0:1] + histT[:, 1:2] + noise_c[...]
    ii = g * _IB + lax.broadcasted_iota(jnp.int32, (_IB, 1), 0)

    def body(jc, acc):
        kj = (hist_a[pl.ds(jc, 1), :] + hist_b[pl.ds(jc, 1), :]
              + noise_r[pl.ds(jc, 1), :])
        jj = jc * _JC + lax.broadcasted_iota(jnp.int32, (1, _JC), 1)
        before = (kj < ki) | ((kj == ki) & (jj < ii))
        cnt = jnp.where(before, 1.0, 0.0)
        return acc + jnp.sum(cnt, axis=1, keepdims=True)

    acc = lax.fori_loop(0, _NP // _JC, body, jnp.zeros((_IB, 1), jnp.float32))
    out_ref[...] = acc.astype(jnp.int32)


_rank_tc = pl.pallas_call(
    _rank_body,
    grid=(_NP // _IB,),
    in_specs=[
        pl.BlockSpec((_NP // _JC, _JC), lambda g: (0, 0)),
        pl.BlockSpec((_NP // _JC, _JC), lambda g: (0, 0)),
        pl.BlockSpec((_NP // _JC, _JC), lambda g: (0, 0)),
        pl.BlockSpec((_IB, 2), lambda g: (g, 0)),
        pl.BlockSpec((_IB, 1), lambda g: (g, 0)),
    ],
    out_specs=pl.BlockSpec((_IB, 1), lambda g: (g, 0)),
    out_shape=jax.ShapeDtypeStruct((_NP, 1), jnp.int32),
)


def _permute_sc_body(x_hbm, rankf_hbm, xs_hbm, idx_hbm,
                     rk_v, rows_v, vals_v, trk_v, trows_v, tvals_v):
    c = lax.axis_index("c")
    s = lax.axis_index("s")
    w = c * 16 + s

    def do_chunk(ci):
        pltpu.sync_copy(rankf_hbm.at[pl.ds(ci * 128, 128)], rk_v)
        pltpu.sync_copy(x_hbm.at[pl.ds(ci * 128, 128), :], rows_v)
        for k in range(8):
            vals_v[pl.ds(k * 16, 16)] = (
                ci * 128 + k * 16 + lax.broadcasted_iota(jnp.int32, (16,), 0))
        pltpu.sync_copy(rows_v, xs_hbm.at[rk_v])
        pltpu.sync_copy(vals_v, idx_hbm.at[rk_v])

    do_chunk(w)
    do_chunk(w + 32)

    @pl.when(w < 14)
    def _():
        do_chunk(w + 64)

    @pl.when(w == 14)
    def _():
        pltpu.sync_copy(rankf_hbm.at[pl.ds(9984, 16)], trk_v)
        pltpu.sync_copy(x_hbm.at[pl.ds(9984, 16), :], trows_v)
        tvals_v[...] = 9984 + lax.broadcasted_iota(jnp.int32, (16,), 0)
        pltpu.sync_copy(trows_v, xs_hbm.at[trk_v])
        pltpu.sync_copy(tvals_v, idx_hbm.at[trk_v])


@functools.lru_cache(maxsize=1)
def _sc_kernels():
    mesh = plsc.VectorSubcoreMesh(core_axis_name="c", subcore_axis_name="s",
                                  num_cores=2, num_subcores=16)
    hist_sc = pl.kernel(
        _hist_sc_body,
        out_type=jax.ShapeDtypeStruct((2, _NP), jnp.float32),
        mesh=mesh,
        scratch_types=[
            pltpu.VMEM((80, 128), jnp.int32),
            pltpu.VMEM((128,), jnp.float32),
            pltpu.VMEM((640,), jnp.float32),
            pltpu.VMEM_SHARED((_NP,), jnp.float32),
        ],
    )
    permute_sc = pl.kernel(
        _permute_sc_body,
        out_type=(jax.ShapeDtypeStruct((_N, _D), jnp.float32),
                  jax.ShapeDtypeStruct((_N,), jnp.int32)),
        mesh=mesh,
        scratch_types=[
            pltpu.VMEM((128,), jnp.int32),
            pltpu.VMEM((128, _D), jnp.float32),
            pltpu.VMEM((128,), jnp.int32),
            pltpu.VMEM((16,), jnp.int32),
            pltpu.VMEM((16, _D), jnp.float32),
            pltpu.VMEM((16,), jnp.int32),
        ],
    )
    return hist_sc, permute_sc


def kernel(x, edge_index):
    hist_sc, permute_sc = _sc_kernels()
    x2 = x if x.ndim == 2 else x[0]
    src = edge_index[0].astype(jnp.int32)
    src_pad = jnp.concatenate(
        [src, jnp.full((_EP - _E,), _PAD_BIN, jnp.int32)]).reshape(32, 80, 128)

    noise = (jax.random.uniform(jax.random.key(1), (1, _N), dtype=jnp.float32)
             * 0.1)
    noise_pad = jnp.concatenate(
        [noise[0], jnp.full((_NP - _N,), jnp.inf, jnp.float32)])

    hist2 = hist_sc(src_pad)
    hist_a = hist2[0].reshape(_NP // _JC, _JC)
    hist_b = hist2[1].reshape(_NP // _JC, _JC)
    noise_r = noise_pad.reshape(_NP // _JC, _JC)
    histT = hist2.T
    noise_c = noise_pad.reshape(_NP, 1)

    ranks = _rank_tc(hist_a, hist_b, noise_r, histT, noise_c)
    rankf = ranks.reshape(_NP)

    xs, sidx = permute_sc(x2, rankf)
    return (xs[None], sidx[None])
